# baseline (speedup 1.0000x reference)
import functools

import jax
import jax.numpy as jnp
from jax import lax
from jax.experimental import pallas as pl
from jax.experimental.pallas import tpu as pltpu
from jax.experimental.pallas import tpu_sc as plsc

_VOCAB = 1000000
_EMBED = 64
_CTX = 20
_HIDDEN = 128
_BATCH = 1024

_NC = 2
_NS = 16
_NW = _NC * _NS
_NTOTAL = _BATCH * _CTX
_BPW = _NTOTAL // _NW
_CHUNK = 128
_NCHUNK = _BPW // _CHUNK


def _gather_body(table_hbm, idx_hbm, out_hbm, idx_v, rows_v, sem):
    wid = lax.axis_index("s") * _NC + lax.axis_index("c")
    pltpu.sync_copy(idx_hbm.at[wid], idx_v)
    copies = []
    for c in range(_NCHUNK):
        copies.append(
            pltpu.async_copy(
                table_hbm.at[idx_v.at[c]],
                rows_v.at[pl.ds(c * _CHUNK, _CHUNK)],
                sem,
            )
        )
    for cp in copies:
        cp.wait()
    pltpu.sync_copy(rows_v, out_hbm.at[pl.ds(wid * _BPW, _BPW)])


@functools.cache
def _make_gather():
    return functools.partial(
        pl.kernel,
        out_type=jax.ShapeDtypeStruct((_NTOTAL, _EMBED), jnp.float32),
        mesh=plsc.VectorSubcoreMesh(core_axis_name="c", subcore_axis_name="s"),
        scratch_types=[
            pltpu.VMEM((_NCHUNK, _CHUNK), jnp.int32),
            pltpu.VMEM((_BPW, _EMBED), jnp.float32),
            pltpu.SemaphoreType.DMA,
        ],
        compiler_params=pltpu.CompilerParams(use_tc_tiling_on_sc=False),
    )(_gather_body)


_VT = 4096


def _mlp_body(e_ref, w1_ref, b1_ref, w2_ref, b2_ref, out_ref, h_ref):
    @pl.when(pl.program_id(0) == 0)
    def _():
        h_ref[...] = jnp.maximum(
            jnp.dot(e_ref[...], w1_ref[...], preferred_element_type=jnp.float32)
            + b1_ref[...],
            0.0,
        )

    out_ref[...] = (
        jnp.dot(h_ref[...], w2_ref[...], preferred_element_type=jnp.float32)
        + b2_ref[...]
    )


def kernel(x, emb, W1, b1, W2, b2):
    idx = x.reshape(_NW, _NCHUNK, _CHUNK)
    e = _make_gather()(emb, idx).reshape(_BATCH, _CTX * _EMBED)

    grid = pl.cdiv(_VOCAB, _VT)
    out = pl.pallas_call(
        _mlp_body,
        grid=(grid,),
        in_specs=[
            pl.BlockSpec((_BATCH, _CTX * _EMBED), lambda i: (0, 0)),
            pl.BlockSpec((_CTX * _EMBED, _HIDDEN), lambda i: (0, 0)),
            pl.BlockSpec((1, _HIDDEN), lambda i: (0, 0)),
            pl.BlockSpec((_HIDDEN, _VT), lambda i: (0, i)),
            pl.BlockSpec((1, _VT), lambda i: (0, i)),
        ],
        out_specs=pl.BlockSpec((_BATCH, _VT), lambda i: (0, i)),
        out_shape=jax.ShapeDtypeStruct((_BATCH, _VOCAB), jnp.float32),
        scratch_shapes=[pltpu.VMEM((_BATCH, _HIDDEN), jnp.float32)],
    )(e, W1, b1.reshape(1, _HIDDEN), W2, b2.reshape(1, _VOCAB))
    return out


# speedup vs baseline: 1.0695x; 1.0695x over previous
import functools

import jax
import jax.numpy as jnp
from jax import lax
from jax.experimental import pallas as pl
from jax.experimental.pallas import tpu as pltpu
from jax.experimental.pallas import tpu_sc as plsc

_VOCAB = 1000000
_EMBED = 64
_CTX = 20
_HIDDEN = 128
_BATCH = 1024

_NC = 2
_NS = 16
_NW = _NC * _NS
_NTOTAL = _BATCH * _CTX
_BPW = _NTOTAL // _NW
_BATW = _BATCH // _NW
_CHUNK = 64
_NCHUNK = _BPW // _CHUNK
_L = 16


def _gather_body(table_hbm, idx_hbm, out_hbm, idx_v, stage_v, e_v, sem):
    wid = lax.axis_index("s") * _NC + lax.axis_index("c")
    pltpu.sync_copy(idx_hbm.at[wid], idx_v.at[pl.ds(0, _BPW // 128)])

    def load_word(g):
        v = idx_v[g // 128, pl.ds(g % 128, _L)]
        return v[0]

    for c in range(_NCHUNK):
        def fire(j, _):
            w = load_word(c * _CHUNK + j)
            pltpu.async_copy(table_hbm.at[w >> 3], stage_v.at[j], sem)
            return 0

        lax.fori_loop(0, _CHUNK, fire, 0)
        pltpu.make_async_copy(
            table_hbm.at[pl.ds(0, _CHUNK)], stage_v, sem
        ).wait()
        def sel(j, _):
            g = c * _CHUNK + j
            w = load_word(g)
            r = w & 7
            b = g // 20
            p = g - b * 20
            for k in range(_EMBED // _L):
                e_v[b, pl.ds(p * _EMBED + k * _L, _L)] = (
                    stage_v[j, r, pl.ds(k * _L, _L)]
                )
            return 0

        lax.fori_loop(0, _CHUNK, sel, 0)
    pltpu.sync_copy(e_v, out_hbm.at[pl.ds(wid * _BATW, _BATW)])


@functools.cache
def _make_gather():
    return functools.partial(
        pl.kernel,
        out_type=jax.ShapeDtypeStruct((_BATCH, _CTX * _EMBED), jnp.float32),
        mesh=plsc.VectorSubcoreMesh(core_axis_name="c", subcore_axis_name="s"),
        scratch_types=[
            pltpu.VMEM((_BPW // 128 + 1, 128), jnp.int32),
            pltpu.VMEM((_CHUNK, 8, _EMBED), jnp.float32),
            pltpu.VMEM((_BATW, _CTX * _EMBED), jnp.float32),
            pltpu.SemaphoreType.DMA,
        ],
        compiler_params=pltpu.CompilerParams(needs_layout_passes=False),
    )(_gather_body)


_VT = 4096


def _mlp_body(e_ref, w1_ref, b1_ref, w2_ref, b2_ref, out_ref, h_ref):
    @pl.when(pl.program_id(0) == 0)
    def _():
        h_ref[...] = jnp.maximum(
            jnp.dot(e_ref[...], w1_ref[...], preferred_element_type=jnp.float32)
            + b1_ref[...],
            0.0,
        )

    out_ref[...] = (
        jnp.dot(h_ref[...], w2_ref[...], preferred_element_type=jnp.float32)
        + b2_ref[...]
    )


def kernel(x, emb, W1, b1, W2, b2):
    table3 = emb.reshape(_VOCAB // 8, 8, _EMBED)
    idx = x.reshape(_NW, _BPW // 128, 128)
    e = _make_gather()(table3, idx)

    grid = pl.cdiv(_VOCAB, _VT)
    out = pl.pallas_call(
        _mlp_body,
        grid=(grid,),
        in_specs=[
            pl.BlockSpec((_BATCH, _CTX * _EMBED), lambda i: (0, 0)),
            pl.BlockSpec((_CTX * _EMBED, _HIDDEN), lambda i: (0, 0)),
            pl.BlockSpec((1, _HIDDEN), lambda i: (0, 0)),
            pl.BlockSpec((_HIDDEN, _VT), lambda i: (0, i)),
            pl.BlockSpec((1, _VT), lambda i: (0, i)),
        ],
        out_specs=pl.BlockSpec((_BATCH, _VT), lambda i: (0, i)),
        out_shape=jax.ShapeDtypeStruct((_BATCH, _VOCAB), jnp.float32),
        scratch_shapes=[pltpu.VMEM((_BATCH, _HIDDEN), jnp.float32)],
    )(e, W1, b1.reshape(1, _HIDDEN), W2, b2.reshape(1, _VOCAB))
    return out


# speedup vs baseline: 3.5503x; 3.3195x over previous
import functools

import jax
import jax.numpy as jnp
from jax import lax
from jax.experimental import pallas as pl
from jax.experimental.pallas import tpu as pltpu
from jax.experimental.pallas import tpu_sc as plsc

_VOCAB = 1000000
_EMBED = 64
_CTX = 20
_HIDDEN = 128
_BATCH = 1024

_NC = 2
_NS = 16
_NW = _NC * _NS
_NTOTAL = _BATCH * _CTX
_BPW = _NTOTAL // _NW
_BATW = _BATCH // _NW
_ROW = _CTX * _EMBED
_L = 16


_CHUNK = 64
_NCHUNK = _BPW // _CHUNK


def _gather_body(table_hbm, idx_hbm, out_hbm, idx_v, stage_v, e_v, sem):
    wid = lax.axis_index("s") * _NC + lax.axis_index("c")
    pltpu.sync_copy(idx_hbm.at[wid], idx_v.at[pl.ds(0, _BPW // 128)])

    def load_word(g):
        v = idx_v[g // 128, pl.ds(g % 128, _L)]
        return v[0]

    for c in range(_NCHUNK):
        def fire(j, _):
            w = load_word(c * _CHUNK + j)
            pltpu.async_copy(table_hbm.at[w >> 3], stage_v.at[j], sem)
            return 0

        lax.fori_loop(0, _CHUNK, fire, 0)
        pltpu.make_async_copy(
            table_hbm.at[pl.ds(0, _CHUNK)], stage_v, sem
        ).wait()
        def sel(j, _):
            g = c * _CHUNK + j
            w = load_word(g)
            r = w & 7
            b = g // 20
            p = g - b * 20
            for k in range(_EMBED // _L):
                e_v[b, pl.ds(p * _EMBED + k * _L, _L)] = (
                    stage_v[j, r, pl.ds(k * _L, _L)]
                )
            return 0

        lax.fori_loop(0, _CHUNK, sel, 0)
    pltpu.sync_copy(e_v, out_hbm.at[pl.ds(wid * _BATW, _BATW)])


@functools.cache
def _make_gather():
    return functools.partial(
        pl.kernel,
        out_type=jax.ShapeDtypeStruct((_BATCH, _ROW), jnp.float32),
        mesh=plsc.VectorSubcoreMesh(core_axis_name="c", subcore_axis_name="s"),
        scratch_types=[
            pltpu.VMEM((_BPW // 128 + 1, 128), jnp.int32),
            pltpu.VMEM((_CHUNK, 8, _EMBED), jnp.float32),
            pltpu.VMEM((_BATW, _ROW), jnp.float32),
            pltpu.SemaphoreType.DMA,
        ],
        compiler_params=pltpu.CompilerParams(
            use_tc_tiling_on_sc=True, needs_layout_passes=False
        ),
    )(_gather_body)


_VT = 4096


def _mlp_body(e_ref, w1_ref, b1_ref, w2t_ref, b2_ref, out_ref, ht_ref):
    @pl.when(pl.program_id(0) == 0)
    def _():
        h = jnp.maximum(
            jnp.dot(e_ref[...], w1_ref[...], preferred_element_type=jnp.float32)
            + b1_ref[...],
            0.0,
        )
        ht_ref[...] = h.T

    out_ref[...] = (
        jnp.dot(w2t_ref[...], ht_ref[...], preferred_element_type=jnp.float32)
        + b2_ref[...].T
    )


def kernel(x, emb, W1, b1, W2, b2):
    table3 = emb.reshape(_VOCAB // 8, 8, _EMBED)
    w2t = W2.T
    idx = x.reshape(_NW, _BPW // 128, 128)
    e = _make_gather()(table3, idx)

    grid = pl.cdiv(_VOCAB, _VT)
    out_t = pl.pallas_call(
        _mlp_body,
        grid=(grid,),
        in_specs=[
            pl.BlockSpec((_BATCH, _ROW), lambda i: (0, 0)),
            pl.BlockSpec((_ROW, _HIDDEN), lambda i: (0, 0)),
            pl.BlockSpec((1, _HIDDEN), lambda i: (0, 0)),
            pl.BlockSpec((_VT, _HIDDEN), lambda i: (i, 0)),
            pl.BlockSpec((1, _VT), lambda i: (0, i)),
        ],
        out_specs=pl.BlockSpec((_VT, _BATCH), lambda i: (i, 0)),
        out_shape=jax.ShapeDtypeStruct((_VOCAB, _BATCH), jnp.float32),
        scratch_shapes=[pltpu.VMEM((_HIDDEN, _BATCH), jnp.float32)],
    )(e, W1, b1.reshape(1, _HIDDEN), w2t, b2.reshape(1, _VOCAB))
    return out_t.T
